# baseline (speedup 1.0000x reference)
import jax
import jax.numpy as jnp
import numpy as np
from jax.experimental import pallas as pl
from jax.experimental.pallas import tpu as pltpu

B, Q, L = 32, 300, 256
P = 256


def _body(idx_ref, logits_ref, boxes_ref, scale_ref, pos_ref,
          ob_ref, os_ref):
    x = logits_ref[0]
    rowmax = jnp.max(x, axis=-1, keepdims=True)
    denom = jnp.sum(jnp.exp(x - rowmax), axis=-1, keepdims=True)

    pos = pos_ref[0] > 0
    xm = jnp.where(pos, x, -jnp.inf)
    m = jnp.max(xm, axis=-1, keepdims=True)
    scores = jnp.exp(m - rowmax) / denom

    bx = boxes_ref[0]
    cx, cy, w, h = bx[:, 0:1], bx[:, 1:2], bx[:, 2:3], bx[:, 3:4]
    xyxy = jnp.concatenate(
        [cx - 0.5 * w, cy - 0.5 * h, cx + 0.5 * w, cy + 0.5 * h], axis=1)
    xyxy = xyxy * scale_ref[0]

    r_iota = jax.lax.broadcasted_iota(jnp.int32, (Q, Q), 0)
    q_iota = jax.lax.broadcasted_iota(jnp.int32, (Q, Q), 1)
    eye = (r_iota == q_iota).astype(jnp.float32)
    s_row = jnp.sum(eye * scores, axis=0, keepdims=True)

    gt = (scores > s_row).astype(jnp.float32)
    eq_lt = jnp.logical_and(scores == s_row, r_iota < q_iota)
    rank = jnp.sum(gt + eq_lt.astype(jnp.float32), axis=0, keepdims=True)

    perm = (rank == r_iota.astype(jnp.float32)).astype(jnp.float32)

    os_ref[0] = jnp.sum(perm * s_row, axis=1, keepdims=True)
    ob_ref[0] = jnp.dot(perm, xyxy, preferred_element_type=jnp.float32)


def kernel(pred_logits, pred_boxes, target_sizes, positive_map, phrase_batch_idx):
    ts = target_sizes.astype(jnp.float32)
    scale = jnp.stack([ts[:, 1], ts[:, 0], ts[:, 1], ts[:, 0]], axis=1)
    scale = scale.reshape(B, 1, 4)
    pos3 = positive_map.reshape(P, 1, L)

    grid_spec = pltpu.PrefetchScalarGridSpec(
        num_scalar_prefetch=1,
        grid=(P,),
        in_specs=[
            pl.BlockSpec((1, Q, L), lambda p, idx: (idx[p], 0, 0)),
            pl.BlockSpec((1, Q, 4), lambda p, idx: (idx[p], 0, 0)),
            pl.BlockSpec((1, 1, 4), lambda p, idx: (idx[p], 0, 0)),
            pl.BlockSpec((1, 1, L), lambda p, idx: (p, 0, 0)),
        ],
        out_specs=[
            pl.BlockSpec((1, Q, 4), lambda p, idx: (p, 0, 0)),
            pl.BlockSpec((1, Q, 1), lambda p, idx: (p, 0, 0)),
        ],
    )
    sorted_boxes, sorted_scores = pl.pallas_call(
        _body,
        grid_spec=grid_spec,
        out_shape=[
            jax.ShapeDtypeStruct((P, Q, 4), jnp.float32),
            jax.ShapeDtypeStruct((P, Q, 1), jnp.float32),
        ],
        compiler_params=pltpu.CompilerParams(
            dimension_semantics=("arbitrary",),
        ),
    )(phrase_batch_idx, pred_logits, pred_boxes, scale, pos3)
    return (sorted_boxes, sorted_scores.reshape(P, Q))
